# Initial kernel scaffold; baseline (speedup 1.0000x reference)
import jax
import jax.numpy as jnp
from jax.experimental import pallas as pl


def kernel(boxes, cls_logits):
    raise NotImplementedError("write your pallas kernel here")


# speedup vs baseline: 2.6902x; 2.6902x over previous
import functools

import jax
import jax.numpy as jnp
from jax.experimental import pallas as pl
from jax.experimental.pallas import tpu as pltpu

_NUM_LABELS = 80
_NCAND = 1000
_NPAD = 1024
_THR = 0.65
_MAX_DETS = 100
_OUT_PAD = 128


def _nms_body(rows_ref, cols_ref, logits_ref, cand_ref, labels_ref,
              boxes_out_ref, scores_out_ref, labels_out_ref,
              s_ref, keep_ref, sig_ref, ksm_ref, cnt_ref):
    sig_ref[...] = 1.0 / (1.0 + jnp.exp(-logits_ref[...]))
    keep_ref[...] = jnp.ones((1, _NPAD), jnp.float32)

    x1r = rows_ref[0:1, :]
    y1r = rows_ref[1:2, :]
    x2r = rows_ref[2:3, :]
    y2r = rows_ref[3:4, :]
    area_r = (x2r - x1r) * (y2r - y1r)
    colj = jax.lax.broadcasted_iota(jnp.int32, (128, _NPAD), 1)

    for rb in range(_NPAD // 128):
        sl = pl.ds(rb * 128, 128)
        x1c = cols_ref[sl, 0:1]
        y1c = cols_ref[sl, 1:2]
        x2c = cols_ref[sl, 2:3]
        y2c = cols_ref[sl, 3:4]
        area_c = (x2c - x1c) * (y2c - y1c)
        w = jnp.maximum(jnp.minimum(x2c, x2r) - jnp.maximum(x1c, x1r), 0.0)
        h = jnp.maximum(jnp.minimum(y2c, y2r) - jnp.maximum(y1c, y1r), 0.0)
        inter = w * h
        union = area_c + area_r - inter
        iou = inter / jnp.maximum(union, 1e-9)
        rowi = jax.lax.broadcasted_iota(jnp.int32, (128, _NPAD), 0) + rb * 128
        cond = (iou > _THR) & (colj > rowi) & (colj < _NCAND) & (rowi < _NCAND)
        s_ref[sl, :] = jnp.where(cond, 1.0, 0.0)

    lane = jax.lax.broadcasted_iota(jnp.int32, (1, _NPAD), 1)

    def nms_step(i, _):
        k_i = jnp.max(jnp.where(lane == i, keep_ref[...], 0.0))
        ksm_ref[i] = jnp.where(k_i > 0.0, 1, 0).astype(jnp.int32)

        @pl.when(k_i > 0.0)
        def _():
            keep_ref[...] = keep_ref[...] * (1.0 - s_ref[pl.ds(i, 1), :])

        return 0

    jax.lax.fori_loop(0, _NCAND, nms_step, 0)

    cnt_ref[0] = 0

    def take_kept(i, _):
        c = cnt_ref[0]

        @pl.when((ksm_ref[i] == 1) & (c < _MAX_DETS))
        def _():
            boxes_out_ref[pl.ds(c, 1), :] = cand_ref[pl.ds(i, 1), :]
            scores_out_ref[pl.ds(c, 1), :] = sig_ref[pl.ds(i, 1), :]
            labels_out_ref[pl.ds(c, 1), :] = labels_ref[pl.ds(i, 1), :]
            cnt_ref[0] = c + 1

        return 0

    jax.lax.fori_loop(0, _NCAND, take_kept, 0)

    def take_suppressed(i, _):
        c = cnt_ref[0]

        @pl.when((ksm_ref[i] == 0) & (c < _MAX_DETS))
        def _():
            boxes_out_ref[pl.ds(c, 1), :] = cand_ref[pl.ds(i, 1), :]
            scores_out_ref[pl.ds(c, 1), :] = jnp.full((1, 1), -1.0, jnp.float32)
            labels_out_ref[pl.ds(c, 1), :] = labels_ref[pl.ds(i, 1), :]
            cnt_ref[0] = c + 1

        return 0

    jax.lax.fori_loop(0, _NCAND, take_suppressed, 0)


@jax.jit
def kernel(boxes, cls_logits):
    flat = cls_logits.reshape(-1)
    top_logits, top_idx = jax.lax.top_k(flat, _NCAND)
    box_ids = top_idx // _NUM_LABELS
    labels = top_idx % _NUM_LABELS
    cand = jnp.take(boxes, box_ids, axis=0)

    pad = _NPAD - _NCAND
    cand_p = jnp.pad(cand, ((0, pad), (0, 0)))
    labels_p = jnp.pad(labels, (0, pad)).astype(jnp.int32)
    logits_p = jnp.pad(top_logits, (0, pad), constant_values=-1e30)

    nms_boxes = cand_p + labels_p.astype(jnp.float32)[:, None] * 4096.0
    rows = nms_boxes.T
    cols = nms_boxes

    out = pl.pallas_call(
        _nms_body,
        out_shape=[
            jax.ShapeDtypeStruct((_OUT_PAD, 4), jnp.float32),
            jax.ShapeDtypeStruct((_OUT_PAD, 1), jnp.float32),
            jax.ShapeDtypeStruct((_OUT_PAD, 1), jnp.int32),
        ],
        scratch_shapes=[
            pltpu.VMEM((_NPAD, _NPAD), jnp.float32),
            pltpu.VMEM((1, _NPAD), jnp.float32),
            pltpu.VMEM((_NPAD, 1), jnp.float32),
            pltpu.SMEM((_NPAD,), jnp.int32),
            pltpu.SMEM((2,), jnp.int32),
        ],
    )(rows, cols, logits_p[:, None], cand_p, labels_p[:, None])

    final_boxes = out[0][:_MAX_DETS]
    final_scores = out[1][:_MAX_DETS, 0]
    final_labels = out[2][:_MAX_DETS, 0]
    return final_boxes, final_scores, final_labels


# speedup vs baseline: 3.7010x; 1.3758x over previous
import functools

import jax
import jax.numpy as jnp
import numpy as np
from jax.experimental import pallas as pl
from jax.experimental.pallas import tpu as pltpu

_NUM_LABELS = 80
_NCAND = 1000
_NPAD = 1024
_THR = 0.65
_MAX_DETS = 100
_OUT_PAD = 128

_NROWS = 1600
_NLANES = 1024
_FLAT_PAD = _NROWS * _NLANES
_MARKER = np.int32(-(2 ** 31))


def _topk_body(flat_ref, boxes_ref, logit_out, cand_out, labels_out,
               u_ref, m1_ref, m2_ref, m3_ref):
    logit_out[...] = jnp.full((_NPAD, 1), -1e30, jnp.float32)
    cand_out[...] = jnp.zeros((_NPAD, 4), jnp.float32)
    labels_out[...] = jnp.zeros((_NPAD, 1), jnp.int32)

    for ch in range(5):
        sl = pl.ds(ch * 320, 320)
        x = flat_ref[sl, :]
        b = jax.lax.bitcast_convert_type(x, jnp.int32)
        key = jnp.where(b < 0, b ^ np.int32(0x7FFFFFFF), b)
        u_ref[sl, :] = key
        m1_ref[sl, :] = jnp.max(key, axis=1, keepdims=True)
        k3 = key.reshape(40, 8, _NLANES)
        m2 = jnp.max(jnp.max(k3, axis=2), axis=1)
        m2_ref[pl.ds(ch * 40, 40), :] = m2.reshape(40, 1)
        k4 = key.reshape(5, 64, _NLANES)
        m3 = jnp.max(jnp.max(k4, axis=2), axis=1)
        m3_ref[pl.ds(ch * 5, 5), :] = m3.reshape(5, 1)

    sub25 = jax.lax.broadcasted_iota(jnp.int32, (25, 1), 0)
    sub8 = jax.lax.broadcasted_iota(jnp.int32, (8, 1), 0)
    lane = jax.lax.broadcasted_iota(jnp.int32, (1, _NLANES), 1)
    big = np.int32(2 ** 30)

    def step(t, _):
        m = jnp.max(m3_ref[...])
        g3 = jnp.min(jnp.where(m3_ref[...] == m, sub25, big))
        grp2 = m2_ref[pl.ds(g3 * 8, 8), :]
        g2 = g3 * 8 + jnp.min(jnp.where(grp2 == m, sub8, big))
        grp1 = m1_ref[pl.ds(g2 * 8, 8), :]
        r = g2 * 8 + jnp.min(jnp.where(grp1 == m, sub8, big))
        row = u_ref[pl.ds(r, 1), :]
        l = jnp.min(jnp.where(row == m, lane, big))
        flat_idx = r * _NLANES + l
        box_id = flat_idx // _NUM_LABELS
        label = flat_idx - box_id * _NUM_LABELS
        vb = jnp.where(m < 0, m ^ np.int32(0x7FFFFFFF), m)
        val = jax.lax.bitcast_convert_type(vb, jnp.float32)
        tsl = pl.ds(t, 1)
        logit_out[tsl, :] = val.reshape(1, 1)
        labels_out[tsl, :] = label.reshape(1, 1)
        cand_out[tsl, :] = boxes_ref[pl.ds(box_id, 1), :]
        nrow = jnp.where(lane == l, _MARKER, row)
        u_ref[pl.ds(r, 1), :] = nrow
        m1_ref[pl.ds(r, 1), :] = jnp.max(nrow).reshape(1, 1)
        nm2 = jnp.max(m1_ref[pl.ds((r // 8) * 8, 8), :])
        m2_ref[pl.ds(r // 8, 1), :] = nm2.reshape(1, 1)
        nm3 = jnp.max(m2_ref[pl.ds((r // 64) * 8, 8), :])
        m3_ref[pl.ds(r // 64, 1), :] = nm3.reshape(1, 1)
        return 0

    jax.lax.fori_loop(0, _NCAND, step, 0)


def _nms_body(rows_ref, cols_ref, logits_ref, cand_ref, labels_ref,
              boxes_out_ref, scores_out_ref, labels_out_ref,
              s_ref, keep_ref, sig_ref, ksm_ref, cnt_ref):
    sig_ref[...] = 1.0 / (1.0 + jnp.exp(-logits_ref[...]))
    keep_ref[...] = jnp.ones((1, _NPAD), jnp.float32)

    x1r = rows_ref[0:1, :]
    y1r = rows_ref[1:2, :]
    x2r = rows_ref[2:3, :]
    y2r = rows_ref[3:4, :]
    area_r = (x2r - x1r) * (y2r - y1r)
    colj = jax.lax.broadcasted_iota(jnp.int32, (128, _NPAD), 1)

    for rb in range(_NPAD // 128):
        sl = pl.ds(rb * 128, 128)
        x1c = cols_ref[sl, 0:1]
        y1c = cols_ref[sl, 1:2]
        x2c = cols_ref[sl, 2:3]
        y2c = cols_ref[sl, 3:4]
        area_c = (x2c - x1c) * (y2c - y1c)
        w = jnp.maximum(jnp.minimum(x2c, x2r) - jnp.maximum(x1c, x1r), 0.0)
        h = jnp.maximum(jnp.minimum(y2c, y2r) - jnp.maximum(y1c, y1r), 0.0)
        inter = w * h
        union = area_c + area_r - inter
        iou = inter / jnp.maximum(union, 1e-9)
        rowi = jax.lax.broadcasted_iota(jnp.int32, (128, _NPAD), 0) + rb * 128
        cond = (iou > _THR) & (colj > rowi) & (colj < _NCAND) & (rowi < _NCAND)
        s_ref[sl, :] = jnp.where(cond, 1.0, 0.0)

    lane = jax.lax.broadcasted_iota(jnp.int32, (1, _NPAD), 1)

    def nms_step(i, _):
        k_i = jnp.max(jnp.where(lane == i, keep_ref[...], 0.0))
        ksm_ref[i] = jnp.where(k_i > 0.0, 1, 0).astype(jnp.int32)

        @pl.when(k_i > 0.0)
        def _():
            keep_ref[...] = keep_ref[...] * (1.0 - s_ref[pl.ds(i, 1), :])

        return 0

    jax.lax.fori_loop(0, _NCAND, nms_step, 0)

    cnt_ref[0] = 0

    def take_kept(i, _):
        c = cnt_ref[0]

        @pl.when((ksm_ref[i] == 1) & (c < _MAX_DETS))
        def _():
            boxes_out_ref[pl.ds(c, 1), :] = cand_ref[pl.ds(i, 1), :]
            scores_out_ref[pl.ds(c, 1), :] = sig_ref[pl.ds(i, 1), :]
            labels_out_ref[pl.ds(c, 1), :] = labels_ref[pl.ds(i, 1), :]
            cnt_ref[0] = c + 1

        return 0

    jax.lax.fori_loop(0, _NCAND, take_kept, 0)

    def take_suppressed(i, _):
        c = cnt_ref[0]

        @pl.when((ksm_ref[i] == 0) & (c < _MAX_DETS))
        def _():
            boxes_out_ref[pl.ds(c, 1), :] = cand_ref[pl.ds(i, 1), :]
            scores_out_ref[pl.ds(c, 1), :] = jnp.full((1, 1), -1.0, jnp.float32)
            labels_out_ref[pl.ds(c, 1), :] = labels_ref[pl.ds(i, 1), :]
            cnt_ref[0] = c + 1

        return 0

    jax.lax.fori_loop(0, _NCAND, take_suppressed, 0)


@jax.jit
def kernel(boxes, cls_logits):
    flat = cls_logits.reshape(-1)
    flat_p = jnp.pad(flat, (0, _FLAT_PAD - flat.shape[0]),
                     constant_values=-jnp.inf).reshape(_NROWS, _NLANES)

    logits_p, cand_p, labels_2d = pl.pallas_call(
        _topk_body,
        out_shape=[
            jax.ShapeDtypeStruct((_NPAD, 1), jnp.float32),
            jax.ShapeDtypeStruct((_NPAD, 4), jnp.float32),
            jax.ShapeDtypeStruct((_NPAD, 1), jnp.int32),
        ],
        scratch_shapes=[
            pltpu.VMEM((_NROWS, _NLANES), jnp.int32),
            pltpu.VMEM((_NROWS, 1), jnp.int32),
            pltpu.VMEM((200, 1), jnp.int32),
            pltpu.VMEM((25, 1), jnp.int32),
        ],
    )(flat_p, boxes)
    labels_p = labels_2d[:, 0]

    nms_boxes = cand_p + labels_p.astype(jnp.float32)[:, None] * 4096.0
    rows = nms_boxes.T
    cols = nms_boxes

    out = pl.pallas_call(
        _nms_body,
        out_shape=[
            jax.ShapeDtypeStruct((_OUT_PAD, 4), jnp.float32),
            jax.ShapeDtypeStruct((_OUT_PAD, 1), jnp.float32),
            jax.ShapeDtypeStruct((_OUT_PAD, 1), jnp.int32),
        ],
        scratch_shapes=[
            pltpu.VMEM((_NPAD, _NPAD), jnp.float32),
            pltpu.VMEM((1, _NPAD), jnp.float32),
            pltpu.VMEM((_NPAD, 1), jnp.float32),
            pltpu.SMEM((_NPAD,), jnp.int32),
            pltpu.SMEM((2,), jnp.int32),
        ],
    )(rows, cols, logits_p, cand_p, labels_2d)

    final_boxes = out[0][:_MAX_DETS]
    final_scores = out[1][:_MAX_DETS, 0]
    final_labels = out[2][:_MAX_DETS, 0]
    return final_boxes, final_scores, final_labels


# speedup vs baseline: 3.7943x; 1.0252x over previous
import functools

import jax
import jax.numpy as jnp
import numpy as np
from jax.experimental import pallas as pl
from jax.experimental.pallas import tpu as pltpu

_NUM_LABELS = 80
_NCAND = 1000
_NPAD = 1024
_THR = 0.65
_MAX_DETS = 100
_OUT_PAD = 128

_NROWS = 1600
_NLANES = 1024
_FLAT_PAD = _NROWS * _NLANES
_MARKER = np.int32(-(2 ** 31))


def _topk_body(flat_ref, boxes_ref, logit_out, cand_out, labels_out,
               u_ref, m1_ref, m2_ref, m3_ref):
    logit_out[...] = jnp.full((_NPAD, 1), -1e30, jnp.float32)
    cand_out[...] = jnp.zeros((_NPAD, 4), jnp.float32)
    labels_out[...] = jnp.zeros((_NPAD, 1), jnp.int32)

    for ch in range(5):
        sl = pl.ds(ch * 320, 320)
        x = flat_ref[sl, :]
        b = jax.lax.bitcast_convert_type(x, jnp.int32)
        key = jnp.where(b < 0, b ^ np.int32(0x7FFFFFFF), b)
        u_ref[sl, :] = key
        m1_ref[sl, :] = jnp.max(key, axis=1, keepdims=True)
        k3 = key.reshape(40, 8, _NLANES)
        m2 = jnp.max(jnp.max(k3, axis=2), axis=1)
        m2_ref[pl.ds(ch * 40, 40), :] = m2.reshape(40, 1)
        k4 = key.reshape(5, 64, _NLANES)
        m3 = jnp.max(jnp.max(k4, axis=2), axis=1)
        m3_ref[pl.ds(ch * 5, 5), :] = m3.reshape(5, 1)

    sub25 = jax.lax.broadcasted_iota(jnp.int32, (25, 1), 0)
    sub8 = jax.lax.broadcasted_iota(jnp.int32, (8, 1), 0)
    lane = jax.lax.broadcasted_iota(jnp.int32, (1, _NLANES), 1)
    big = np.int32(2 ** 30)

    def step(t, _):
        m = jnp.max(m3_ref[...])
        g3 = jnp.min(jnp.where(m3_ref[...] == m, sub25, big))
        grp2 = m2_ref[pl.ds(g3 * 8, 8), :]
        g2 = g3 * 8 + jnp.min(jnp.where(grp2 == m, sub8, big))
        grp1 = m1_ref[pl.ds(g2 * 8, 8), :]
        r = g2 * 8 + jnp.min(jnp.where(grp1 == m, sub8, big))
        row = u_ref[pl.ds(r, 1), :]
        l = jnp.min(jnp.where(row == m, lane, big))
        flat_idx = r * _NLANES + l
        box_id = flat_idx // _NUM_LABELS
        label = flat_idx - box_id * _NUM_LABELS
        vb = jnp.where(m < 0, m ^ np.int32(0x7FFFFFFF), m)
        val = jax.lax.bitcast_convert_type(vb, jnp.float32)
        tsl = pl.ds(t, 1)
        logit_out[tsl, :] = val.reshape(1, 1)
        labels_out[tsl, :] = label.reshape(1, 1)
        cand_out[tsl, :] = boxes_ref[pl.ds(box_id, 1), :]
        nrow = jnp.where(lane == l, _MARKER, row)
        u_ref[pl.ds(r, 1), :] = nrow
        m1_ref[pl.ds(r, 1), :] = jnp.max(nrow).reshape(1, 1)
        nm2 = jnp.max(m1_ref[pl.ds((r // 8) * 8, 8), :])
        m2_ref[pl.ds(r // 8, 1), :] = nm2.reshape(1, 1)
        nm3 = jnp.max(m2_ref[pl.ds((r // 64) * 8, 8), :])
        m3_ref[pl.ds(r // 64, 1), :] = nm3.reshape(1, 1)
        return 0

    jax.lax.fori_loop(0, _NCAND, step, 0)


def _nms_body(rows_ref, cols_ref, logits_ref, cand_ref, labels_ref,
              boxes_out_ref, scores_out_ref, labels_out_ref,
              s_ref, u_ref, keep_ref):
    keep_ref[...] = jnp.ones((1, _NPAD), jnp.float32)

    x1r = rows_ref[0:1, :]
    y1r = rows_ref[1:2, :]
    x2r = rows_ref[2:3, :]
    y2r = rows_ref[3:4, :]
    area_r = (x2r - x1r) * (y2r - y1r)
    colj = jax.lax.broadcasted_iota(jnp.int32, (128, _NPAD), 1)

    for rb in range(_NPAD // 128):
        sl = pl.ds(rb * 128, 128)
        x1c = cols_ref[sl, 0:1]
        y1c = cols_ref[sl, 1:2]
        x2c = cols_ref[sl, 2:3]
        y2c = cols_ref[sl, 3:4]
        area_c = (x2c - x1c) * (y2c - y1c)
        w = jnp.maximum(jnp.minimum(x2c, x2r) - jnp.maximum(x1c, x1r), 0.0)
        h = jnp.maximum(jnp.minimum(y2c, y2r) - jnp.maximum(y1c, y1r), 0.0)
        inter = w * h
        union = area_c + area_r - inter
        iou = inter / jnp.maximum(union, 1e-9)
        rowi = jax.lax.broadcasted_iota(jnp.int32, (128, _NPAD), 0) + rb * 128
        cond = (iou > _THR) & (colj > rowi) & (colj < _NCAND) & (rowi < _NCAND)
        s_ref[sl, :] = jnp.where(cond, 1.0, 0.0)
        u_ref[sl, :] = jnp.where(colj > rowi, 1.0, 0.0)

    lane = jax.lax.broadcasted_iota(jnp.int32, (1, _NPAD), 1)

    def nms_step(i, _):
        keep = keep_ref[...]
        kbc = jnp.max(jnp.where(lane == i, keep, 0.0), axis=1, keepdims=True)
        keep_ref[...] = keep * (1.0 - kbc * s_ref[pl.ds(i, 1), :])
        return 0

    jax.lax.fori_loop(0, _NCAND, nms_step, 0)

    keepf = keep_ref[...] * jnp.where(lane < _NCAND, 1.0, 0.0)
    hi = jax.lax.Precision.HIGHEST
    rank = jax.lax.dot_general(keepf, u_ref[...],
                               (((1,), (0,)), ((), ())), precision=hi)
    nk = jnp.sum(keepf, axis=1, keepdims=True)
    lane_f = lane.astype(jnp.float32)
    pos = keepf * rank + (1.0 - keepf) * (nk + lane_f - rank)

    p_col = jax.lax.broadcasted_iota(
        jnp.int32, (_OUT_PAD, 1), 0).astype(jnp.float32)
    onehot = jnp.where(pos == p_col, 1.0, 0.0)
    boxes_out_ref[...] = jax.lax.dot_general(
        onehot, cand_ref[...], (((1,), (0,)), ((), ())), precision=hi)
    sig = 1.0 / (1.0 + jnp.exp(-logits_ref[...]))
    sel_sig = jax.lax.dot_general(
        onehot, sig, (((1,), (0,)), ((), ())), precision=hi)
    scores_out_ref[...] = jnp.where(p_col < nk, sel_sig, -1.0)
    lab_f = labels_ref[...].astype(jnp.float32)
    sel_lab = jax.lax.dot_general(
        onehot, lab_f, (((1,), (0,)), ((), ())), precision=hi)
    labels_out_ref[...] = sel_lab.astype(jnp.int32)


@jax.jit
def kernel(boxes, cls_logits):
    flat = cls_logits.reshape(-1)
    flat_p = jnp.pad(flat, (0, _FLAT_PAD - flat.shape[0]),
                     constant_values=-jnp.inf).reshape(_NROWS, _NLANES)

    logits_p, cand_p, labels_2d = pl.pallas_call(
        _topk_body,
        out_shape=[
            jax.ShapeDtypeStruct((_NPAD, 1), jnp.float32),
            jax.ShapeDtypeStruct((_NPAD, 4), jnp.float32),
            jax.ShapeDtypeStruct((_NPAD, 1), jnp.int32),
        ],
        scratch_shapes=[
            pltpu.VMEM((_NROWS, _NLANES), jnp.int32),
            pltpu.VMEM((_NROWS, 1), jnp.int32),
            pltpu.VMEM((200, 1), jnp.int32),
            pltpu.VMEM((25, 1), jnp.int32),
        ],
    )(flat_p, boxes)
    labels_p = labels_2d[:, 0]

    nms_boxes = cand_p + labels_p.astype(jnp.float32)[:, None] * 4096.0
    rows = nms_boxes.T
    cols = nms_boxes

    out = pl.pallas_call(
        _nms_body,
        out_shape=[
            jax.ShapeDtypeStruct((_OUT_PAD, 4), jnp.float32),
            jax.ShapeDtypeStruct((_OUT_PAD, 1), jnp.float32),
            jax.ShapeDtypeStruct((_OUT_PAD, 1), jnp.int32),
        ],
        scratch_shapes=[
            pltpu.VMEM((_NPAD, _NPAD), jnp.float32),
            pltpu.VMEM((_NPAD, _NPAD), jnp.float32),
            pltpu.VMEM((1, _NPAD), jnp.float32),
        ],
    )(rows, cols, logits_p, cand_p, labels_2d)

    final_boxes = out[0][:_MAX_DETS]
    final_scores = out[1][:_MAX_DETS, 0]
    final_labels = out[2][:_MAX_DETS, 0]
    return final_boxes, final_scores, final_labels


# speedup vs baseline: 6.5886x; 1.7365x over previous
import functools

import jax
import jax.numpy as jnp
import numpy as np
from jax.experimental import pallas as pl
from jax.experimental.pallas import tpu as pltpu

_NUM_LABELS = 80
_NCAND = 1000
_NPAD = 1024
_THR = 0.65
_MAX_DETS = 100
_OUT_PAD = 128

_NROWS = 1600
_NLANES = 1024
_FLAT_PAD = _NROWS * _NLANES
_MARKER = np.int32(-(2 ** 31))


def _topk_body(flat_ref, boxes_ref, logit_out, cand_out, labels_out,
               u_ref, m1_ref, m2_ref, flat_scr):
    logit_out[...] = jnp.full((_NPAD, 1), -1e30, jnp.float32)
    flat_scr[...] = jnp.zeros((_NPAD, 1), jnp.int32)

    for ch in range(5):
        sl = pl.ds(ch * 320, 320)
        x = flat_ref[sl, :]
        b = jax.lax.bitcast_convert_type(x, jnp.int32)
        key = jnp.where(b < 0, b ^ np.int32(0x7FFFFFFF), b)
        u_ref[sl, :] = key
        m1_ref[sl, :] = jnp.max(key, axis=1, keepdims=True)
        k3 = key.reshape(40, 8, _NLANES)
        m2 = jnp.max(jnp.max(k3, axis=2), axis=1)
        m2_ref[pl.ds(ch * 40, 40), :] = m2.reshape(40, 1)

    sub200 = jax.lax.broadcasted_iota(jnp.int32, (200, 1), 0)
    sub8 = jax.lax.broadcasted_iota(jnp.int32, (8, 1), 0)
    lane = jax.lax.broadcasted_iota(jnp.int32, (1, _NLANES), 1)
    big = np.int32(2 ** 30)

    def step(t, _):
        m2v = m2_ref[...]
        mv = jnp.max(m2v, axis=0, keepdims=True)
        g2 = jnp.min(jnp.where(m2v == mv, sub200, big))
        grp1 = m1_ref[pl.ds(g2 * 8, 8), :]
        r = g2 * 8 + jnp.min(jnp.where(grp1 == mv, sub8, big))
        row = u_ref[pl.ds(r, 1), :]
        lv = jnp.min(jnp.where(row == mv, lane, big), axis=1, keepdims=True)
        mvr = jnp.max(row, axis=1, keepdims=True)
        vb = jnp.where(mvr < 0, mvr ^ np.int32(0x7FFFFFFF), mvr)
        tsl = pl.ds(t, 1)
        logit_out[tsl, :] = jax.lax.bitcast_convert_type(vb, jnp.float32)
        flat_scr[tsl, :] = r * _NLANES + lv
        nrow = jnp.where(lane == lv, _MARKER, row)
        u_ref[pl.ds(r, 1), :] = nrow
        m1_ref[pl.ds(r, 1), :] = jnp.max(nrow, axis=1, keepdims=True)
        nm2 = jnp.max(m1_ref[pl.ds(g2 * 8, 8), :], axis=0, keepdims=True)
        m2_ref[pl.ds(g2, 1), :] = nm2
        return 0

    jax.lax.fori_loop(0, _NCAND, step, 0)

    fc = flat_scr[...]
    bf = fc.astype(jnp.float32)
    box_id = jnp.floor((bf + 0.5) * (1.0 / _NUM_LABELS)).astype(jnp.int32)
    labels_out[...] = fc - box_id * _NUM_LABELS
    valid = jax.lax.broadcasted_iota(jnp.int32, (_NPAD, 1), 0) < _NCAND
    labels_out[...] = jnp.where(valid, labels_out[...], 0)
    bid_f = jnp.where(valid, box_id, -1).astype(jnp.float32)
    b_row = jax.lax.broadcasted_iota(jnp.int32, (1, _NPAD), 1)
    hi = jax.lax.Precision.HIGHEST
    acc = jnp.zeros((_NPAD, 4), jnp.float32)
    for c in range(20):
        oh = jnp.where(bid_f == (b_row + c * _NPAD).astype(jnp.float32),
                       1.0, 0.0)
        acc = acc + jax.lax.dot_general(
            oh, boxes_ref[pl.ds(c * _NPAD, _NPAD), :],
            (((1,), (0,)), ((), ())), precision=hi)
    cand_out[...] = acc


def _nms_body(rows_ref, cols_ref, logits_ref, cand_ref, labels_ref,
              boxes_out_ref, scores_out_ref, labels_out_ref,
              s_ref, u_ref, keep_ref):
    keep_ref[...] = jnp.ones((1, _NPAD), jnp.float32)

    x1r = rows_ref[0:1, :]
    y1r = rows_ref[1:2, :]
    x2r = rows_ref[2:3, :]
    y2r = rows_ref[3:4, :]
    area_r = (x2r - x1r) * (y2r - y1r)
    colj = jax.lax.broadcasted_iota(jnp.int32, (128, _NPAD), 1)

    for rb in range(_NPAD // 128):
        sl = pl.ds(rb * 128, 128)
        x1c = cols_ref[sl, 0:1]
        y1c = cols_ref[sl, 1:2]
        x2c = cols_ref[sl, 2:3]
        y2c = cols_ref[sl, 3:4]
        area_c = (x2c - x1c) * (y2c - y1c)
        w = jnp.maximum(jnp.minimum(x2c, x2r) - jnp.maximum(x1c, x1r), 0.0)
        h = jnp.maximum(jnp.minimum(y2c, y2r) - jnp.maximum(y1c, y1r), 0.0)
        inter = w * h
        union = area_c + area_r - inter
        iou = inter / jnp.maximum(union, 1e-9)
        rowi = jax.lax.broadcasted_iota(jnp.int32, (128, _NPAD), 0) + rb * 128
        cond = (iou > _THR) & (colj > rowi) & (colj < _NCAND) & (rowi < _NCAND)
        s_ref[sl, :] = jnp.where(cond, 1.0, 0.0)
        u_ref[sl, :] = jnp.where(colj > rowi, 1.0, 0.0)

    lane = jax.lax.broadcasted_iota(jnp.int32, (1, _NPAD), 1)

    def nms_step(i, _):
        keep = keep_ref[...]
        kbc = jnp.max(jnp.where(lane == i, keep, 0.0), axis=1, keepdims=True)
        keep_ref[...] = keep * (1.0 - kbc * s_ref[pl.ds(i, 1), :])
        return 0

    jax.lax.fori_loop(0, _NCAND, nms_step, 0)

    keepf = keep_ref[...] * jnp.where(lane < _NCAND, 1.0, 0.0)
    hi = jax.lax.Precision.HIGHEST
    rank = jax.lax.dot_general(keepf, u_ref[...],
                               (((1,), (0,)), ((), ())), precision=hi)
    nk = jnp.sum(keepf, axis=1, keepdims=True)
    lane_f = lane.astype(jnp.float32)
    pos = keepf * rank + (1.0 - keepf) * (nk + lane_f - rank)

    p_col = jax.lax.broadcasted_iota(
        jnp.int32, (_OUT_PAD, 1), 0).astype(jnp.float32)
    onehot = jnp.where(pos == p_col, 1.0, 0.0)
    boxes_out_ref[...] = jax.lax.dot_general(
        onehot, cand_ref[...], (((1,), (0,)), ((), ())), precision=hi)
    sig = 1.0 / (1.0 + jnp.exp(-logits_ref[...]))
    sel_sig = jax.lax.dot_general(
        onehot, sig, (((1,), (0,)), ((), ())), precision=hi)
    scores_out_ref[...] = jnp.where(p_col < nk, sel_sig, -1.0)
    lab_f = labels_ref[...].astype(jnp.float32)
    sel_lab = jax.lax.dot_general(
        onehot, lab_f, (((1,), (0,)), ((), ())), precision=hi)
    labels_out_ref[...] = sel_lab.astype(jnp.int32)


@jax.jit
def kernel(boxes, cls_logits):
    flat = cls_logits.reshape(-1)
    flat_p = jnp.pad(flat, (0, _FLAT_PAD - flat.shape[0]),
                     constant_values=-jnp.inf).reshape(_NROWS, _NLANES)

    logits_p, cand_p, labels_2d = pl.pallas_call(
        _topk_body,
        out_shape=[
            jax.ShapeDtypeStruct((_NPAD, 1), jnp.float32),
            jax.ShapeDtypeStruct((_NPAD, 4), jnp.float32),
            jax.ShapeDtypeStruct((_NPAD, 1), jnp.int32),
        ],
        scratch_shapes=[
            pltpu.VMEM((_NROWS, _NLANES), jnp.int32),
            pltpu.VMEM((_NROWS, 1), jnp.int32),
            pltpu.VMEM((200, 1), jnp.int32),
            pltpu.VMEM((_NPAD, 1), jnp.int32),
        ],
    )(flat_p, jnp.pad(boxes, ((0, 20 * _NPAD - boxes.shape[0]), (0, 0))))
    labels_p = labels_2d[:, 0]

    nms_boxes = cand_p + labels_p.astype(jnp.float32)[:, None] * 4096.0
    rows = nms_boxes.T
    cols = nms_boxes

    out = pl.pallas_call(
        _nms_body,
        out_shape=[
            jax.ShapeDtypeStruct((_OUT_PAD, 4), jnp.float32),
            jax.ShapeDtypeStruct((_OUT_PAD, 1), jnp.float32),
            jax.ShapeDtypeStruct((_OUT_PAD, 1), jnp.int32),
        ],
        scratch_shapes=[
            pltpu.VMEM((_NPAD, _NPAD), jnp.float32),
            pltpu.VMEM((_NPAD, _NPAD), jnp.float32),
            pltpu.VMEM((1, _NPAD), jnp.float32),
        ],
    )(rows, cols, logits_p, cand_p, labels_2d)

    final_boxes = out[0][:_MAX_DETS]
    final_scores = out[1][:_MAX_DETS, 0]
    final_labels = out[2][:_MAX_DETS, 0]
    return final_boxes, final_scores, final_labels
